# baseline (speedup 1.0000x reference)
import functools
import jax
import jax.numpy as jnp
from jax import lax
from jax.experimental import pallas as pl
from jax.experimental.pallas import tpu as pltpu
from jax.experimental.pallas import tpu_sc as plsc

VOCAB_ROWS = 1000000
D = 64
NC = 2
NS = 16
NW = NC * NS
NBLK = VOCAB_ROWS // 128
NMAIN = (NBLK // NW) * NW
CHUNK = 128

_params = pltpu.CompilerParams(
    use_tc_tiling_on_sc=True, needs_layout_passes=False)
_mesh = plsc.VectorSubcoreMesh(core_axis_name="c", subcore_axis_name="s")


def _iota16():
    return lax.iota(jnp.int32, 16)


def _relayout_body(wt_hbm, wtail_hbm, wlin_hbm, tbuf, rbuf, isem, osem):
    wid = lax.axis_index("s") * NC + lax.axis_index("c")
    iota = _iota16()
    zero = iota - iota
    cvecs = [iota + 16 * j for j in range(4)]

    def load(slot, j):
        return pltpu.make_async_copy(
            wt_hbm.at[pl.ds(0, 64),
                      pl.ds(pl.multiple_of((wid + NW * j) * 128, 128), 128)],
            tbuf.at[slot], isem.at[slot])

    def store(slot, j):
        return pltpu.make_async_copy(
            rbuf.at[slot],
            wlin_hbm.at[pl.ds(pl.multiple_of((wid + NW * j) * 128, 128), 128)],
            osem.at[slot])

    def transpose(slot):
        tb = tbuf.at[slot]
        rb = rbuf.at[slot]
        for r in range(128):
            rsplat = zero + r
            for j in range(4):
                vals = plsc.load_gather(tb, [cvecs[j], rsplat])
                rb[r, pl.ds(16 * j, 16)] = vals

    NPAIR = NMAIN // NW // 2

    load(0, 0).start()
    load(1, 1).start()

    def body(m, _):
        for par in (0, 1):
            j = 2 * m + par
            load(par, j).wait()

            @pl.when(m > 0)
            def _drain():
                store(par, j - 2).wait()

            transpose(par)

            @pl.when(m < NPAIR - 1)
            def _next():
                load(par, j + 2).start()

            store(par, j).start()
        return _

    lax.fori_loop(0, NPAIR, body, None)
    store(0, 2 * NPAIR - 2).wait()
    store(1, 2 * NPAIR - 1).wait()

    @pl.when(wid < 4)
    def _tail():
        off = pl.multiple_of((NMAIN + wid) * 128, 128)
        pltpu.make_async_copy(
            wt_hbm.at[pl.ds(0, 64), pl.ds(off, 128)], tbuf.at[0],
            isem.at[0]).start()
        pltpu.make_async_copy(
            wt_hbm.at[pl.ds(0, 64), pl.ds(off, 128)], tbuf.at[0],
            isem.at[0]).wait()
        transpose(0)
        pltpu.make_async_copy(
            rbuf.at[0], wlin_hbm.at[pl.ds(off, 128)], osem.at[0]).start()
        pltpu.make_async_copy(
            rbuf.at[0], wlin_hbm.at[pl.ds(off, 128)], osem.at[0]).wait()

    NTAIL2 = NBLK * 128
    @pl.when(wid == 4)
    def _tail2():
        pltpu.make_async_copy(
            wtail_hbm, rbuf.at[0, pl.ds(0, 64)], isem.at[0]).start()
        pltpu.make_async_copy(
            wtail_hbm, rbuf.at[0, pl.ds(0, 64)], isem.at[0]).wait()
        pltpu.make_async_copy(
            rbuf.at[0, pl.ds(0, 64)], wlin_hbm.at[pl.ds(NTAIL2, 64)],
            osem.at[0]).start()
        pltpu.make_async_copy(
            rbuf.at[0, pl.ds(0, 64)], wlin_hbm.at[pl.ds(NTAIL2, 64)],
            osem.at[0]).wait()


def _gather_body(xb_hbm, wlin_hbm, out_hbm, idx_v, rows_v, tile_v, gsem, ssem):
    wid = lax.axis_index("s") * NC + lax.axis_index("c")
    n = idx_v.shape[0]
    base = wid * n
    pltpu.sync_copy(xb_hbm.at[pl.ds(pl.multiple_of(base, 8), n)], idx_v)
    iota = _iota16()
    zero = iota - iota
    bvecs = [iota + 16 * k for k in range(8)]

    def gather(slot, g):
        return pltpu.make_async_copy(
            wlin_hbm.at[idx_v.at[g]], rows_v.at[slot], gsem.at[slot])

    def stores(slot, g):
        blk = base + g
        t = blk // 32
        bg = lax.rem(blk, 32)
        return [pltpu.make_async_copy(
            tile_v.at[slot, cg], out_hbm.at[t, cg, bg], ssem.at[slot])
            for cg in range(8)]

    def transpose(slot):
        rv = rows_v.at[slot]
        tv = tile_v.at[slot]
        for cg in range(8):
            for cs in range(8):
                cvec = zero + (8 * cg + cs)
                for k in range(8):
                    vals = plsc.load_gather(rv, [bvecs[k], cvec])
                    tv[cg, cs, pl.ds(16 * k, 16)] = vals

    gather(0, 0).start()
    gather(1, 1).start()

    def body(m, _):
        for par in (0, 1):
            g = 2 * m + par

            @pl.when(m > 0)
            def _drain():
                for d in stores(par, g - 2):
                    d.wait()

            gather(par, g).wait()
            transpose(par)

            @pl.when(m < n // 2 - 1)
            def _next():
                gather(par, g + 2).start()

            for d in stores(par, g):
                d.start()
        return _

    lax.fori_loop(0, n // 2, body, None)
    for d in stores(0, n - 2):
        d.wait()
    for d in stores(1, n - 1):
        d.wait()


_relayout = pl.kernel(
    _relayout_body,
    out_type=jax.ShapeDtypeStruct((VOCAB_ROWS, 128), jnp.float32),
    mesh=_mesh,
    scratch_types=[
        pltpu.VMEM((2, 64, 128), jnp.float32),
        pltpu.VMEM((2, 128, 128), jnp.float32),
        pltpu.SemaphoreType.DMA((2,)),
        pltpu.SemaphoreType.DMA((2,)),
    ],
    compiler_params=_params,
)

_gather = pl.kernel(
    _gather_body,
    out_type=jax.ShapeDtypeStruct((200, 8, 32, 8, 128), jnp.float32),
    mesh=_mesh,
    scratch_types=[
        pltpu.VMEM((6400 // NW, CHUNK), jnp.int32),
        pltpu.VMEM((2, CHUNK, 128), jnp.float32),
        pltpu.VMEM((2, 8, 8, 128), jnp.float32),
        pltpu.SemaphoreType.DMA((2,)),
        pltpu.SemaphoreType.DMA((2,)),
    ],
    compiler_params=_params,
)


@jax.jit
def kernel(x, weight):
    xb = x.astype(jnp.int32).T.reshape(6400, 128)
    wtail = jnp.pad(weight[NBLK * 128:], ((0, 0), (0, 64)))
    w_lin = _relayout(weight.T, wtail)
    out5 = _gather(xb, w_lin)
    return jnp.transpose(out5, (2, 4, 0, 1, 3)).reshape(4096, 200, D)


# speedup vs baseline: 2.0696x; 2.0696x over previous
import functools
import jax
import jax.numpy as jnp
from jax import lax
from jax.experimental import pallas as pl
from jax.experimental.pallas import tpu as pltpu
from jax.experimental.pallas import tpu_sc as plsc

VOCAB_ROWS = 1000000
D = 64
NC = 2
NS = 16
NW = NC * NS
NBLK = VOCAB_ROWS // 128
NMAIN = (NBLK // NW) * NW
CHUNK = 128

_params = pltpu.CompilerParams(
    use_tc_tiling_on_sc=True, needs_layout_passes=False,
    disable_bounds_checks=True)
_mesh = plsc.VectorSubcoreMesh(core_axis_name="c", subcore_axis_name="s")


def _iota16():
    return lax.iota(jnp.int32, 16)


def _relayout_body(wt_hbm, wtail_hbm, wlin_hbm, tbuf, rbuf, isem, osem):
    wid = lax.axis_index("s") * NC + lax.axis_index("c")
    iota = _iota16()
    zero = iota - iota
    cvecs = [iota + 16 * j for j in range(4)]

    def load(slot, j):
        return pltpu.make_async_copy(
            wt_hbm.at[pl.ds(0, 64),
                      pl.ds(pl.multiple_of((wid + NW * j) * 128, 128), 128)],
            tbuf.at[slot], isem.at[slot])

    def store(slot, j):
        return pltpu.make_async_copy(
            rbuf.at[slot],
            wlin_hbm.at[pl.ds(pl.multiple_of((wid + NW * j) * 128, 128), 128)],
            osem.at[slot])

    def transpose(slot):
        tb = tbuf.at[slot]
        rb = rbuf.at[slot]

        @plsc.parallel_loop(0, 128, unroll=4)
        def _rows(r):
            rsplat = zero + r
            for j in range(4):
                vals = plsc.load_gather(tb, [cvecs[j], rsplat])
                rb[r, pl.ds(16 * j, 16)] = vals

    NPAIR = NMAIN // NW // 2

    load(0, 0).start()
    load(1, 1).start()

    def body(m, _):
        for par in (0, 1):
            j = 2 * m + par
            load(par, j).wait()

            @pl.when(m > 0)
            def _drain():
                store(par, j - 2).wait()

            transpose(par)

            @pl.when(m < NPAIR - 1)
            def _next():
                load(par, j + 2).start()

            store(par, j).start()
        return _

    lax.fori_loop(0, NPAIR, body, None)
    store(0, 2 * NPAIR - 2).wait()
    store(1, 2 * NPAIR - 1).wait()

    @pl.when(wid < 4)
    def _tail():
        off = pl.multiple_of((NMAIN + wid) * 128, 128)
        pltpu.make_async_copy(
            wt_hbm.at[pl.ds(0, 64), pl.ds(off, 128)], tbuf.at[0],
            isem.at[0]).start()
        pltpu.make_async_copy(
            wt_hbm.at[pl.ds(0, 64), pl.ds(off, 128)], tbuf.at[0],
            isem.at[0]).wait()
        transpose(0)
        pltpu.make_async_copy(
            rbuf.at[0], wlin_hbm.at[pl.ds(off, 128)], osem.at[0]).start()
        pltpu.make_async_copy(
            rbuf.at[0], wlin_hbm.at[pl.ds(off, 128)], osem.at[0]).wait()

    NTAIL2 = NBLK * 128
    @pl.when(wid == 4)
    def _tail2():
        pltpu.make_async_copy(
            wtail_hbm, rbuf.at[0, pl.ds(0, 64)], isem.at[0]).start()
        pltpu.make_async_copy(
            wtail_hbm, rbuf.at[0, pl.ds(0, 64)], isem.at[0]).wait()
        pltpu.make_async_copy(
            rbuf.at[0, pl.ds(0, 64)], wlin_hbm.at[pl.ds(NTAIL2, 64)],
            osem.at[0]).start()
        pltpu.make_async_copy(
            rbuf.at[0, pl.ds(0, 64)], wlin_hbm.at[pl.ds(NTAIL2, 64)],
            osem.at[0]).wait()


def _gather_body(xb_hbm, wlin_hbm, out_hbm, idx_v, rows_v, tile_v, gsem, ssem):
    wid = lax.axis_index("s") * NC + lax.axis_index("c")
    n = idx_v.shape[0]
    base = wid * n
    pltpu.sync_copy(xb_hbm.at[pl.ds(pl.multiple_of(base, 8), n)], idx_v)
    iota = _iota16()
    zero = iota - iota
    bvecs = [iota + 16 * k for k in range(8)]

    def gather(slot, g):
        return pltpu.make_async_copy(
            wlin_hbm.at[idx_v.at[g]], rows_v.at[slot], gsem.at[slot])

    def stores(slot, g):
        blk = base + g
        t = blk // 32
        bg = lax.rem(blk, 32)
        return [pltpu.make_async_copy(
            tile_v.at[slot, cg], out_hbm.at[t, cg, bg], ssem.at[slot])
            for cg in range(8)]

    def transpose(slot):
        rv = rows_v.at[slot]
        tv = tile_v.at[slot]

        @plsc.parallel_loop(0, 64, unroll=4)
        def _chans(c):
            cg = c // 8
            cs = lax.rem(c, 8)
            cvec = zero + c
            for k in range(8):
                vals = plsc.load_gather(rv, [bvecs[k], cvec])
                tv[cg, cs, pl.ds(16 * k, 16)] = vals

    gather(0, 0).start()
    gather(1, 1).start()

    def body(m, _):
        for par in (0, 1):
            g = 2 * m + par

            @pl.when(m > 0)
            def _drain():
                for d in stores(par, g - 2):
                    d.wait()

            gather(par, g).wait()
            transpose(par)

            @pl.when(m < n // 2 - 1)
            def _next():
                gather(par, g + 2).start()

            for d in stores(par, g):
                d.start()
        return _

    lax.fori_loop(0, n // 2, body, None)
    for d in stores(0, n - 2):
        d.wait()
    for d in stores(1, n - 1):
        d.wait()


_relayout = pl.kernel(
    _relayout_body,
    out_type=jax.ShapeDtypeStruct((VOCAB_ROWS, 128), jnp.float32),
    mesh=_mesh,
    scratch_types=[
        pltpu.VMEM((2, 64, 128), jnp.float32),
        pltpu.VMEM((2, 128, 128), jnp.float32),
        pltpu.SemaphoreType.DMA((2,)),
        pltpu.SemaphoreType.DMA((2,)),
    ],
    compiler_params=_params,
)

_gather = pl.kernel(
    _gather_body,
    out_type=jax.ShapeDtypeStruct((200, 8, 32, 8, 128), jnp.float32),
    mesh=_mesh,
    scratch_types=[
        pltpu.VMEM((6400 // NW, CHUNK), jnp.int32),
        pltpu.VMEM((2, CHUNK, 128), jnp.float32),
        pltpu.VMEM((2, 8, 8, 128), jnp.float32),
        pltpu.SemaphoreType.DMA((2,)),
        pltpu.SemaphoreType.DMA((2,)),
    ],
    compiler_params=_params,
)


@jax.jit
def kernel(x, weight):
    xb = x.astype(jnp.int32).T.reshape(6400, 128)
    wtail = jnp.pad(weight[NBLK * 128:], ((0, 0), (0, 64)))
    w_lin = _relayout(weight.T, wtail)
    out5 = _gather(xb, w_lin)
    return jnp.transpose(out5, (2, 4, 0, 1, 3)).reshape(4096, 200, D)
